# Initial kernel scaffold; baseline (speedup 1.0000x reference)
import jax
import jax.numpy as jnp
from jax.experimental import pallas as pl


def kernel(x, edge_index, W, b, gamma, beta):
    raise NotImplementedError("write your pallas kernel here")


# speedup vs baseline: 5.5654x; 5.5654x over previous
import functools

import jax
import jax.numpy as jnp
from jax import lax
from jax.experimental import pallas as pl
from jax.experimental.pallas import tpu as pltpu, tpu_sc as plsc

N = 10000
E = 160000
D = 256
DH = D // 2
NC = 2
NS = 16
EPT = E // NS
C = 80
NCH = EPT // C
RPT = 624
TAIL = N - NS * RPT
RCP = 40
NPC = (RPT + TAIL) // RCP

_mesh = plsc.VectorSubcoreMesh(
    core_axis_name="c", subcore_axis_name="s", num_cores=NC, num_subcores=NS)


@functools.partial(
    pl.kernel,
    out_type=jax.ShapeDtypeStruct((NC * N,), jnp.float32),
    mesh=_mesh,
    scratch_types=[
        pltpu.VMEM((NCH, C), jnp.int32),
        pltpu.VMEM((C,), jnp.float32),
        pltpu.VMEM((RPT + TAIL,), jnp.float32),
        pltpu.VMEM_SHARED((N,), jnp.float32),
    ],
)
def _sc_degrees(idx_all, deg_out, e_v, ones_v, zb, deg_sh):
    c = lax.axis_index("c")
    s = lax.axis_index("s")
    pltpu.sync_copy(idx_all.at[c * NS + s], e_v)

    for k in range(C // 16):
        ones_v[pl.ds(k * 16, 16)] = jnp.ones((16,), jnp.float32)

    def _fillz(i, _):
        zb[pl.ds(i * 16, 16)] = jnp.zeros((16,), jnp.float32)
        return 0
    lax.fori_loop(0, (RPT + TAIL) // 16, _fillz, 0)

    pltpu.sync_copy(zb, deg_sh.at[pl.ds(s * RPT, RPT + TAIL)])
    plsc.subcore_barrier()

    def _chunk(j, _):
        pltpu.sync_copy(ones_v, deg_sh.at[e_v.at[j]], add=True)
        return 0
    lax.fori_loop(0, NCH, _chunk, 0)
    plsc.subcore_barrier()

    pltpu.sync_copy(deg_sh.at[pl.ds(s * RPT, RPT + TAIL)], zb)
    pltpu.sync_copy(zb, deg_out.at[pl.ds(c * N + s * RPT, RPT + TAIL)])


@functools.partial(
    pl.kernel,
    out_type=jax.ShapeDtypeStruct((NC * N, DH), jnp.float32),
    mesh=_mesh,
    scratch_types=[
        pltpu.VMEM((NCH, C), jnp.int32),
        pltpu.VMEM((NCH, C), jnp.int32),
        pltpu.VMEM((C, DH), jnp.float32),
        pltpu.VMEM((RCP, DH), jnp.float32),
        pltpu.VMEM_SHARED((N, DH), jnp.float32),
        pltpu.SemaphoreType.DMA,
    ],
)
def _sc_aggregate(h, idx_all, out, src_v, dst_v, rows_v, cb, acc_sh, sem):
    c = lax.axis_index("c")
    s = lax.axis_index("s")
    pltpu.sync_copy(idx_all.at[c * 2 * NS + s], src_v)
    pltpu.sync_copy(idx_all.at[NS + s], dst_v)

    def _fillz(i, _):
        for k in range(DH // 16):
            cb[i, k * 16:(k + 1) * 16] = jnp.zeros((16,), jnp.float32)
        return 0
    lax.fori_loop(0, RCP, _fillz, 0)

    def _zpiece(p, _):
        pltpu.sync_copy(cb, acc_sh.at[pl.ds(s * RPT + p * RCP, RCP)])
        return 0
    lax.fori_loop(0, NPC, _zpiece, 0)
    plsc.subcore_barrier()

    def _chunk(j, _):
        pltpu.async_copy(h.at[src_v.at[j]], rows_v, sem).wait()
        pltpu.sync_copy(rows_v, acc_sh.at[dst_v.at[j]], add=True)
        return 0
    lax.fori_loop(0, NCH, _chunk, 0)
    plsc.subcore_barrier()

    def _piece(p, _):
        pltpu.sync_copy(acc_sh.at[pl.ds(s * RPT + p * RCP, RCP)], cb)
        pltpu.sync_copy(cb, out.at[pl.ds(c * N + s * RPT + p * RCP, RCP)])
        return 0
    lax.fori_loop(0, NPC, _piece, 0)


_BR = 1000
_GRID = N // _BR


def _norm_from_deg(deg_col):
    return jnp.where(deg_col > 0.0,
                     lax.rsqrt(jnp.maximum(deg_col, 1.0)), 0.0)


def _tc_matmul_body(x_ref, w_ref, deg_ref, h_ref):
    norm = _norm_from_deg(deg_ref[0])
    hb = jnp.dot(x_ref[...] * norm, w_ref[...],
                 preferred_element_type=jnp.float32)
    h_ref[0] = hb[:, :DH]
    h_ref[1] = hb[:, DH:]


def _tc_matmul(x, w, hist):
    return pl.pallas_call(
        _tc_matmul_body,
        grid=(_GRID,),
        in_specs=[
            pl.BlockSpec((_BR, D), lambda i: (i, 0)),
            pl.BlockSpec((D, D), lambda i: (0, 0)),
            pl.BlockSpec((1, _BR, 1), lambda i: (0, i, 0)),
        ],
        out_specs=pl.BlockSpec((NC, _BR, DH), lambda i: (0, i, 0)),
        out_shape=jax.ShapeDtypeStruct((NC, N, DH), jnp.float32),
    )(x, w, hist)


def _tc_stats_body(agg_ref, hist_ref, b_ref, sums_ref):
    i = pl.program_id(0)
    a = jnp.concatenate([agg_ref[0], agg_ref[1]], axis=1)
    norm = _norm_from_deg(hist_ref[0])
    a = a * norm + b_ref[...]

    @pl.when(i == 0)
    def _():
        sums_ref[...] = jnp.zeros_like(sums_ref)

    sums_ref[0, :] += jnp.sum(a, axis=0)
    sums_ref[1, :] += jnp.sum(a * a, axis=0)


def _tc_stats(agg, hist, b2):
    return pl.pallas_call(
        _tc_stats_body,
        grid=(_GRID,),
        in_specs=[
            pl.BlockSpec((NC, _BR, DH), lambda i: (0, i, 0)),
            pl.BlockSpec((1, _BR, 1), lambda i: (1, i, 0)),
            pl.BlockSpec((1, D), lambda i: (0, 0)),
        ],
        out_specs=pl.BlockSpec((2, D), lambda i: (0, 0)),
        out_shape=jax.ShapeDtypeStruct((2, D), jnp.float32),
    )(agg, hist, b2)


def _tc_apply_body(agg_ref, hist_ref, sums_ref, x_ref, b_ref, g_ref, bt_ref,
                   out_ref):
    a = jnp.concatenate([agg_ref[0], agg_ref[1]], axis=1)
    norm = _norm_from_deg(hist_ref[0])
    a = a * norm + b_ref[...]
    mean = sums_ref[0:1, :] * (1.0 / N)
    var = sums_ref[1:2, :] * (1.0 / N) - mean * mean
    inv = lax.rsqrt(var + 1e-5)
    hbn = (a - mean) * inv * g_ref[...] + bt_ref[...]
    out_ref[...] = x_ref[...] + jnp.maximum(hbn, 0.0)


def _tc_apply(agg, hist, sums, x, b2, g2, bt2):
    return pl.pallas_call(
        _tc_apply_body,
        grid=(_GRID,),
        in_specs=[
            pl.BlockSpec((NC, _BR, DH), lambda i: (0, i, 0)),
            pl.BlockSpec((1, _BR, 1), lambda i: (1, i, 0)),
            pl.BlockSpec((2, D), lambda i: (0, 0)),
            pl.BlockSpec((_BR, D), lambda i: (i, 0)),
            pl.BlockSpec((1, D), lambda i: (0, 0)),
            pl.BlockSpec((1, D), lambda i: (0, 0)),
            pl.BlockSpec((1, D), lambda i: (0, 0)),
        ],
        out_specs=pl.BlockSpec((_BR, D), lambda i: (i, 0)),
        out_shape=jax.ShapeDtypeStruct((N, D), jnp.float32),
    )(agg, hist, sums, x, b2, g2, bt2)


def kernel(x, edge_index, W, b, gamma, beta):
    e32 = edge_index.astype(jnp.int32)
    er = e32.reshape(NC, NS, NCH, C)
    idx_all = jnp.concatenate([er[0], er[1], er[0] + N], axis=0)
    b2 = b.reshape(1, D).astype(jnp.float32)
    g2 = gamma.reshape(1, D).astype(jnp.float32)
    bt2 = beta.reshape(1, D).astype(jnp.float32)

    hist = _sc_degrees(idx_all).reshape(NC, N, 1)
    h = _tc_matmul(x, W, hist)
    agg = _sc_aggregate(h.reshape(NC * N, DH), idx_all).reshape(NC, N, DH)
    sums = _tc_stats(agg, hist, b2)
    return _tc_apply(agg, hist, sums, x, b2, g2, bt2)


# speedup vs baseline: 7.4860x; 1.3451x over previous
import functools

import jax
import jax.numpy as jnp
from jax import lax
from jax.experimental import pallas as pl
from jax.experimental.pallas import tpu as pltpu, tpu_sc as plsc

N = 10000
E = 160000
D = 256
DH = D // 2
NC = 2
NS = 16
EPT = E // NS
C = 80
NCH = EPT // C
CA = 125
NCHA = EPT // CA
WCH = 8
NW = NCHA // WCH
RPT = 624
TAIL = N - NS * RPT
RCP = 32
NPC = (RPT + TAIL) // RCP

_mesh = plsc.VectorSubcoreMesh(
    core_axis_name="c", subcore_axis_name="s", num_cores=NC, num_subcores=NS)


@functools.partial(
    pl.kernel,
    out_type=jax.ShapeDtypeStruct((NC * N,), jnp.float32),
    mesh=_mesh,
    scratch_types=[
        pltpu.VMEM((NCH, C), jnp.int32),
        pltpu.VMEM((C,), jnp.float32),
        pltpu.VMEM((RPT + TAIL,), jnp.float32),
        pltpu.VMEM_SHARED((N,), jnp.float32),
    ],
)
def _sc_degrees(idx_all, deg_out, e_v, ones_v, zb, deg_sh):
    c = lax.axis_index("c")
    s = lax.axis_index("s")
    pltpu.sync_copy(idx_all.at[c * NS + s], e_v)

    for k in range(C // 16):
        ones_v[pl.ds(k * 16, 16)] = jnp.ones((16,), jnp.float32)

    def _fillz(i, _):
        zb[pl.ds(i * 16, 16)] = jnp.zeros((16,), jnp.float32)
        return 0
    lax.fori_loop(0, (RPT + TAIL) // 16, _fillz, 0)

    pltpu.sync_copy(zb, deg_sh.at[pl.ds(s * RPT, RPT + TAIL)])
    plsc.subcore_barrier()

    def _chunk(j, _):
        pltpu.sync_copy(ones_v, deg_sh.at[e_v.at[j]], add=True)
        return 0
    lax.fori_loop(0, NCH, _chunk, 0)
    plsc.subcore_barrier()

    pltpu.sync_copy(deg_sh.at[pl.ds(s * RPT, RPT + TAIL)], zb)
    pltpu.sync_copy(zb, deg_out.at[pl.ds(c * N + s * RPT, RPT + TAIL)])


@functools.partial(
    pl.kernel,
    out_type=jax.ShapeDtypeStruct((NC * N, DH), jnp.float32),
    mesh=_mesh,
    scratch_types=[
        pltpu.VMEM((WCH, CA), jnp.int32),
        pltpu.VMEM((WCH, CA), jnp.int32),
        pltpu.VMEM((CA, DH), jnp.float32),
        pltpu.VMEM((CA, DH), jnp.float32),
        pltpu.VMEM((RCP, DH), jnp.float32),
        pltpu.VMEM_SHARED((N, DH), jnp.float32),
        pltpu.SemaphoreType.DMA,
        pltpu.SemaphoreType.DMA,
    ],
)
def _sc_aggregate(h, idx_agg, out, srcw, dstw, rows0, rows1, cb, acc_sh,
                  sem0, sem1):
    c = lax.axis_index("c")
    s = lax.axis_index("s")

    def _fillz(i, _):
        for k in range(DH // 16):
            cb[i, k * 16:(k + 1) * 16] = jnp.zeros((16,), jnp.float32)
        return 0
    lax.fori_loop(0, RCP, _fillz, 0)

    def _zpiece(p, _):
        pltpu.sync_copy(cb, acc_sh.at[pl.ds(s * RPT + p * RCP, RCP)])
        return 0
    lax.fori_loop(0, NPC, _zpiece, 0)
    plsc.subcore_barrier()

    def _win(w, _):
        pltpu.sync_copy(idx_agg.at[(c * 2 * NS + s) * NW + w], srcw)
        pltpu.sync_copy(idx_agg.at[(NS + s) * NW + w], dstw)
        pltpu.async_copy(h.at[srcw.at[0]], rows0, sem0)
        pltpu.async_copy(h.at[srcw.at[1]], rows1, sem1)

        def _pair(k, _2):
            pltpu.make_async_copy(h.at[srcw.at[2 * k]], rows0, sem0).wait()
            pltpu.sync_copy(rows0, acc_sh.at[dstw.at[2 * k]], add=True)
            pltpu.async_copy(h.at[srcw.at[2 * k + 2]], rows0, sem0)
            pltpu.make_async_copy(h.at[srcw.at[2 * k + 1]], rows1, sem1).wait()
            pltpu.sync_copy(rows1, acc_sh.at[dstw.at[2 * k + 1]], add=True)
            pltpu.async_copy(h.at[srcw.at[2 * k + 3]], rows1, sem1)
            return 0
        lax.fori_loop(0, WCH // 2 - 1, _pair, 0)
        pltpu.make_async_copy(h.at[srcw.at[WCH - 2]], rows0, sem0).wait()
        pltpu.sync_copy(rows0, acc_sh.at[dstw.at[WCH - 2]], add=True)
        pltpu.make_async_copy(h.at[srcw.at[WCH - 1]], rows1, sem1).wait()
        pltpu.sync_copy(rows1, acc_sh.at[dstw.at[WCH - 1]], add=True)
        return 0
    lax.fori_loop(0, NW, _win, 0)
    plsc.subcore_barrier()

    def _piece(p, _):
        pltpu.sync_copy(acc_sh.at[pl.ds(s * RPT + p * RCP, RCP)], cb)
        pltpu.sync_copy(cb, out.at[pl.ds(c * N + s * RPT + p * RCP, RCP)])
        return 0
    lax.fori_loop(0, NPC, _piece, 0)


_BR = 1000
_GRID = N // _BR


def _norm_from_deg(deg_col):
    return jnp.where(deg_col > 0.0,
                     lax.rsqrt(jnp.maximum(deg_col, 1.0)), 0.0)


def _tc_matmul_body(x_ref, w_ref, deg_ref, h_ref):
    norm = _norm_from_deg(deg_ref[0])
    hb = jnp.dot(x_ref[...] * norm, w_ref[...],
                 preferred_element_type=jnp.float32)
    h_ref[0] = hb[:, :DH]
    h_ref[1] = hb[:, DH:]


def _tc_matmul(x, w, hist):
    return pl.pallas_call(
        _tc_matmul_body,
        grid=(_GRID,),
        in_specs=[
            pl.BlockSpec((_BR, D), lambda i: (i, 0)),
            pl.BlockSpec((D, D), lambda i: (0, 0)),
            pl.BlockSpec((1, _BR, 1), lambda i: (0, i, 0)),
        ],
        out_specs=pl.BlockSpec((NC, _BR, DH), lambda i: (0, i, 0)),
        out_shape=jax.ShapeDtypeStruct((NC, N, DH), jnp.float32),
    )(x, w, hist)


def _tc_stats_body(agg_ref, hist_ref, b_ref, sums_ref):
    i = pl.program_id(0)
    a = jnp.concatenate([agg_ref[0], agg_ref[1]], axis=1)
    norm = _norm_from_deg(hist_ref[0])
    a = a * norm + b_ref[...]

    @pl.when(i == 0)
    def _():
        sums_ref[...] = jnp.zeros_like(sums_ref)

    sums_ref[0, :] += jnp.sum(a, axis=0)
    sums_ref[1, :] += jnp.sum(a * a, axis=0)


def _tc_stats(agg, hist, b2):
    return pl.pallas_call(
        _tc_stats_body,
        grid=(_GRID,),
        in_specs=[
            pl.BlockSpec((NC, _BR, DH), lambda i: (0, i, 0)),
            pl.BlockSpec((1, _BR, 1), lambda i: (1, i, 0)),
            pl.BlockSpec((1, D), lambda i: (0, 0)),
        ],
        out_specs=pl.BlockSpec((2, D), lambda i: (0, 0)),
        out_shape=jax.ShapeDtypeStruct((2, D), jnp.float32),
    )(agg, hist, b2)


def _tc_apply_body(agg_ref, hist_ref, sums_ref, x_ref, b_ref, g_ref, bt_ref,
                   out_ref):
    a = jnp.concatenate([agg_ref[0], agg_ref[1]], axis=1)
    norm = _norm_from_deg(hist_ref[0])
    a = a * norm + b_ref[...]
    mean = sums_ref[0:1, :] * (1.0 / N)
    var = sums_ref[1:2, :] * (1.0 / N) - mean * mean
    inv = lax.rsqrt(var + 1e-5)
    hbn = (a - mean) * inv * g_ref[...] + bt_ref[...]
    out_ref[...] = x_ref[...] + jnp.maximum(hbn, 0.0)


def _tc_apply(agg, hist, sums, x, b2, g2, bt2):
    return pl.pallas_call(
        _tc_apply_body,
        grid=(_GRID,),
        in_specs=[
            pl.BlockSpec((NC, _BR, DH), lambda i: (0, i, 0)),
            pl.BlockSpec((1, _BR, 1), lambda i: (1, i, 0)),
            pl.BlockSpec((2, D), lambda i: (0, 0)),
            pl.BlockSpec((_BR, D), lambda i: (i, 0)),
            pl.BlockSpec((1, D), lambda i: (0, 0)),
            pl.BlockSpec((1, D), lambda i: (0, 0)),
            pl.BlockSpec((1, D), lambda i: (0, 0)),
        ],
        out_specs=pl.BlockSpec((_BR, D), lambda i: (i, 0)),
        out_shape=jax.ShapeDtypeStruct((N, D), jnp.float32),
    )(agg, hist, sums, x, b2, g2, bt2)


def kernel(x, edge_index, W, b, gamma, beta):
    e32 = edge_index.astype(jnp.int32)
    er = e32.reshape(NC, NS, NCH, C)
    idx_all = jnp.concatenate([er[0], er[1]], axis=0)
    era = e32.reshape(NC, NS, NCHA, CA)
    idx_agg = jnp.concatenate([era[0], era[1], era[0] + N],
                              axis=0).reshape(48 * NW, WCH, CA)
    b2 = b.reshape(1, D).astype(jnp.float32)
    g2 = gamma.reshape(1, D).astype(jnp.float32)
    bt2 = beta.reshape(1, D).astype(jnp.float32)

    hist = _sc_degrees(idx_all).reshape(NC, N, 1)
    h = _tc_matmul(x, W, hist)
    agg = _sc_aggregate(h.reshape(NC * N, DH), idx_agg).reshape(NC, N, DH)
    sums = _tc_stats(agg, hist, b2)
    return _tc_apply(agg, hist, sums, x, b2, g2, bt2)
